# Initial kernel scaffold; baseline (speedup 1.0000x reference)
import jax
import jax.numpy as jnp
from jax.experimental import pallas as pl
from jax.experimental.pallas import tpu as pltpu
from jax.experimental.pallas import tpu_sc as plsc

NUM_COLS = 26
VOCAB = 1000
D_MODEL = 64
BATCH = 4096
STRIDE = 1024
TOTAL = BATCH * NUM_COLS
WINDOW = 128


def _combine_body(x_cat_ref, tables_ref, col_ref, comb_ref, idx_ref):
    comb_ref[: VOCAB + 1, :] = tables_ref[0] + col_ref[0]

    @pl.when(pl.program_id(0) == 0)
    def _():
        col_ids = jax.lax.broadcasted_iota(jnp.int32, (BATCH, NUM_COLS), 1)
        idx_ref[...] = x_cat_ref[...] + col_ids * STRIDE


_combine = pl.pallas_call(
    _combine_body,
    grid=(NUM_COLS,),
    in_specs=[
        pl.BlockSpec((BATCH, NUM_COLS), lambda c: (0, 0)),
        pl.BlockSpec((1, VOCAB + 1, D_MODEL), lambda c: (c, 0, 0)),
        pl.BlockSpec((1, 1, D_MODEL), lambda c: (c, 0, 0)),
    ],
    out_specs=[
        pl.BlockSpec((STRIDE, D_MODEL), lambda c: (c, 0)),
        pl.BlockSpec((BATCH, NUM_COLS), lambda c: (0, 0)),
    ],
    out_shape=[
        jax.ShapeDtypeStruct((NUM_COLS * STRIDE, D_MODEL), jnp.float32),
        jax.ShapeDtypeStruct((BATCH, NUM_COLS), jnp.int32),
    ],
)

_mesh = plsc.VectorSubcoreMesh(core_axis_name="c", subcore_axis_name="s")


@pl.kernel(
    out_type=jax.ShapeDtypeStruct((TOTAL, D_MODEL), jnp.float32),
    mesh=_mesh,
)
def _sc_gather(table_hbm, idx_hbm, out_hbm):
    def body(i_vmem, o_vmem):
        pltpu.sync_copy(table_hbm.at[i_vmem.at[0]], o_vmem)

    pltpu.emit_pipeline(
        body,
        grid=(TOTAL // WINDOW,),
        in_specs=[pl.BlockSpec((1, WINDOW), index_map=lambda i: (0, i))],
        out_specs=[pl.BlockSpec((WINDOW, D_MODEL), index_map=lambda i: (i, 0))],
        core_axis_name=("c", "s"),
        dimension_semantics=(pltpu.PARALLEL,),
    )(idx_hbm, out_hbm)


def kernel(x_cat, tables, col_type):
    comb, idx = _combine(
        x_cat.astype(jnp.int32), tables, col_type.reshape(NUM_COLS, 1, D_MODEL)
    )
    flat = _sc_gather(comb, idx.reshape(1, TOTAL))
    return flat.reshape(BATCH, NUM_COLS, D_MODEL)


# speedup vs baseline: 11.4939x; 11.4939x over previous
import jax
import jax.numpy as jnp
from jax.experimental import pallas as pl
from jax.experimental.pallas import tpu as pltpu
from jax.experimental.pallas import tpu_sc as plsc

NUM_COLS = 26
VOCAB = 1000
D_MODEL = 64
BATCH = 4096
STRIDE = 1024
TOTAL = BATCH * NUM_COLS
WINDOW = 128


def _combine_body(x_cat_ref, tables_ref, col_ref, comb_ref, idx_ref):
    comb_ref[: VOCAB + 1, :] = tables_ref[0] + col_ref[0]

    @pl.when(pl.program_id(0) == 0)
    def _():
        col_ids = jax.lax.broadcasted_iota(jnp.int32, (BATCH, NUM_COLS), 1)
        idx_ref[...] = x_cat_ref[...] + col_ids * STRIDE


_combine = pl.pallas_call(
    _combine_body,
    grid=(NUM_COLS,),
    in_specs=[
        pl.BlockSpec((BATCH, NUM_COLS), lambda c: (0, 0)),
        pl.BlockSpec((1, VOCAB + 1, D_MODEL), lambda c: (c, 0, 0)),
        pl.BlockSpec((1, 1, D_MODEL), lambda c: (c, 0, 0)),
    ],
    out_specs=[
        pl.BlockSpec((STRIDE, D_MODEL), lambda c: (c, 0)),
        pl.BlockSpec((BATCH, NUM_COLS), lambda c: (0, 0)),
    ],
    out_shape=[
        jax.ShapeDtypeStruct((NUM_COLS * STRIDE, D_MODEL), jnp.float32),
        jax.ShapeDtypeStruct((BATCH, NUM_COLS), jnp.int32),
    ],
)

_mesh = plsc.VectorSubcoreMesh(core_axis_name="c", subcore_axis_name="s")


@pl.kernel(
    out_type=jax.ShapeDtypeStruct((TOTAL, D_MODEL), jnp.float32),
    mesh=_mesh,
    compiler_params=pltpu.CompilerParams(use_tc_tiling_on_sc=False),
)
def _sc_gather(table_hbm, idx_hbm, out_hbm):
    def body(i_vmem, o_vmem):
        pltpu.sync_copy(table_hbm.at[i_vmem.at[0]], o_vmem)

    pltpu.emit_pipeline(
        body,
        grid=(TOTAL // WINDOW,),
        in_specs=[pl.BlockSpec((1, WINDOW), index_map=lambda i: (0, i))],
        out_specs=[pl.BlockSpec((WINDOW, D_MODEL), index_map=lambda i: (i, 0))],
        core_axis_name=("c", "s"),
        dimension_semantics=(pltpu.PARALLEL,),
    )(idx_hbm, out_hbm)


def kernel(x_cat, tables, col_type):
    comb, idx = _combine(
        x_cat.astype(jnp.int32), tables, col_type.reshape(NUM_COLS, 1, D_MODEL)
    )
    flat = _sc_gather(comb, idx.reshape(1, TOTAL))
    return flat.reshape(BATCH, NUM_COLS, D_MODEL)
